# baseline (speedup 1.0000x reference)
import functools

import jax
import jax.numpy as jnp
from jax import lax
from jax.experimental import pallas as pl
from jax.experimental.pallas import tpu as pltpu
from jax.experimental.pallas import tpu_sc as plsc

TILE = 512
RT = 256
D = 1024
E = 8
N = 4096
A = 2 * N
P = A + E * RT
NT_G = P // RT
NW = 32
CHUNK = A // NW
LN_EPS = 1e-5


def _i32_eq(a, b):
    return 1 - jnp.minimum(jnp.abs(a - b), 1)


def _ln_rows(h, gamma, beta):
    mu = jnp.mean(h, axis=-1, keepdims=True)
    var = jnp.mean((h - mu) ** 2, axis=-1, keepdims=True)
    return gamma * (h - mu) * lax.rsqrt(var + LN_EPS) + beta


def _mem_router_body(x_ref, wm_ref, bm_ref, g_ref, b_ref, wg_ref, bg_ref,
                     tri_ref, xo_ref, ei_ref, ew_ref, rk_ref, cnt_ref,
                     carry_ref):
    t = pl.program_id(0)
    x = x_ref[...]
    gamma = g_ref[...]
    beta = b_ref[...]
    for c in range(2):
        h = lax.dot_general(x, wm_ref[c], (((1,), (1,)), ((), ())),
                            preferred_element_type=jnp.float32)
        h = h + bm_ref[c][None, :]
        x = x + _ln_rows(h, gamma, beta)
    xo_ref[...] = x

    logits = jnp.dot(x, wg_ref[...], preferred_element_type=jnp.float32)
    logits = logits + bg_ref[...]
    ii = lax.broadcasted_iota(jnp.int32, logits.shape, 1)
    v1 = jnp.max(logits, axis=1, keepdims=True)
    i1 = jnp.min(jnp.where(logits == v1, ii, E), axis=1, keepdims=True)
    l2 = jnp.where(ii == i1, -jnp.inf, logits)
    v2 = jnp.max(l2, axis=1, keepdims=True)
    i2 = jnp.min(jnp.where(l2 == v2, ii, E), axis=1, keepdims=True)
    g = jnp.exp(v2 - v1)
    w1 = 1.0 / (1.0 + g)
    w2 = g / (1.0 + g)
    ei_ref[...] = jnp.concatenate([i1, i2], axis=1)
    ew_ref[...] = jnp.concatenate([w1, w2], axis=1)

    o0 = jnp.where(ii == i1, 1.0, 0.0)
    o1 = jnp.where(ii == i2, 1.0, 0.0)

    @pl.when(t == 0)
    def _():
        carry_ref[...] = jnp.zeros_like(carry_ref)

    r0 = carry_ref[0]
    r1 = carry_ref[1]
    tri = tri_ref[...]
    x0 = lax.dot_general(tri, o0, (((1,), (0,)), ((), ())),
                         preferred_element_type=jnp.float32) + r0[None, :]
    x1 = lax.dot_general(tri, o1, (((1,), (0,)), ((), ())),
                         preferred_element_type=jnp.float32) + r1[None, :]
    rank0 = jnp.sum(o0 * (x0 + x1), axis=1, keepdims=True)
    rank1 = jnp.sum(o1 * (x0 + x1 + o0), axis=1, keepdims=True)
    rk_ref[...] = jnp.concatenate([rank0, rank1], axis=1).astype(jnp.int32)
    r0n = r0 + jnp.sum(o0, axis=0)
    r1n = r1 + jnp.sum(o1, axis=0)
    carry_ref[0] = r0n
    carry_ref[1] = r1n
    cnt_ref[...] = (r0n + r1n)[None, :]


def _sc1_body(e_hbm, rk_hbm, st_hbm, x_hbm, p_hbm, xg_hbm,
              ev, rv, sv, pbuf, tbuf, pidx, rows, sem):
    nc = 2
    wid = lax.axis_index("s") * nc + lax.axis_index("c")
    base_a = wid * CHUNK
    lanes = lax.broadcasted_iota(jnp.int32, (16,), 0)

    pltpu.sync_copy(e_hbm.at[pl.ds(base_a, CHUNK)], ev)
    pltpu.sync_copy(rk_hbm.at[pl.ds(base_a, CHUNK)], rv)
    pltpu.sync_copy(st_hbm, sv)

    for v in range(CHUNK // 16):
        evv = ev[pl.ds(v * 16, 16)]
        p = rv[pl.ds(v * 16, 16)]
        for e in range(E):
            p = p + _i32_eq(evv, e) * sv[e]
        pbuf[pl.ds(v * 16, 16)] = p
    pltpu.sync_copy(pbuf, p_hbm.at[pl.ds(base_a, CHUNK)])

    for ch in range(CHUNK // 64):
        a0 = base_a + ch * 64
        for v in range(4):
            tbuf[pl.ds(v * 16, 16)] = (a0 + v * 16 + lanes) >> 1
            pidx[pl.ds(v * 16, 16)] = pbuf[pl.ds(ch * 64 + v * 16, 16)]
        pltpu.async_copy(x_hbm.at[tbuf], rows, sem).wait()
        pltpu.async_copy(rows, xg_hbm.at[pidx], sem).wait()


def _grouped_body(gid_ref, xg_ref, we_ref, be_ref, yg_ref):
    xb = xg_ref[...].astype(jnp.bfloat16)
    y = lax.dot_general(xb, we_ref[0], (((1,), (1,)), ((), ())),
                        preferred_element_type=jnp.float32)
    yg_ref[...] = y + be_ref[0]


def _sc2_body(yg_hbm, p_hbm, l_hbm, pidx, rows, sem):
    nc = 2
    wid = lax.axis_index("s") * nc + lax.axis_index("c")
    base_a = wid * CHUNK
    for ch in range(CHUNK // 64):
        a0 = base_a + ch * 64
        pltpu.sync_copy(p_hbm.at[pl.ds(a0, 64)], pidx)
        pltpu.async_copy(yg_hbm.at[pidx], rows, sem).wait()
        pltpu.sync_copy(rows, l_hbm.at[pl.ds(a0, 64)])


def _fit_body(l_ref, x_ref, ew_ref, wf_ref, bf_ref, g_ref, b_ref, o_ref):
    w0 = ew_ref[:, 0:1]
    w1 = ew_ref[:, 1:2]
    l0 = (l_ref[:, :D] * w0).astype(jnp.bfloat16)
    l1 = (l_ref[:, D:] * w1).astype(jnp.bfloat16)
    o = lax.dot_general(l0, wf_ref[0], (((1,), (0,)), ((), ())),
                        preferred_element_type=jnp.float32)
    o += lax.dot_general(l1, wf_ref[1], (((1,), (0,)), ((), ())),
                         preferred_element_type=jnp.float32)
    o += lax.dot_general(x_ref[...].astype(jnp.bfloat16), wf_ref[2],
                         (((1,), (0,)), ((), ())),
                         preferred_element_type=jnp.float32)
    o += bf_ref[...][None, :]
    o = _ln_rows(o, g_ref[...], b_ref[...])
    o_ref[...] = jax.nn.gelu(o, approximate=True)


def kernel(x, W_mem, b_mem, gamma, beta, W_g, b_g, W_e, b_e, W_fit, b_fit,
           choices):
    Bx, Sx, Dx = x.shape
    nt = N // TILE
    xf = x.reshape(N, Dx)
    wf3 = W_fit.reshape(3, Dx, Dx).astype(jnp.bfloat16)
    we_b = W_e.astype(jnp.bfloat16)
    tri = jnp.tril(jnp.ones((TILE, TILE), jnp.float32), -1)

    full = lambda *s: pl.BlockSpec(s, lambda *_: tuple(0 for _ in s))
    row = pl.BlockSpec((TILE, Dx), lambda t: (t, 0))
    two = pl.BlockSpec((TILE, 2), lambda t: (t, 0))

    xo, eidx, ew, rk, counts = pl.pallas_call(
        _mem_router_body,
        grid=(nt,),
        in_specs=[row, full(2, Dx, Dx), full(2, Dx), full(Dx), full(Dx),
                  full(Dx, E), full(E), full(TILE, TILE)],
        out_specs=[row, two, two, two,
                   pl.BlockSpec((1, E), lambda t: (0, 0))],
        out_shape=[jax.ShapeDtypeStruct((N, Dx), jnp.float32),
                   jax.ShapeDtypeStruct((N, 2), jnp.int32),
                   jax.ShapeDtypeStruct((N, 2), jnp.float32),
                   jax.ShapeDtypeStruct((N, 2), jnp.int32),
                   jax.ShapeDtypeStruct((1, E), jnp.float32)],
        scratch_shapes=[pltpu.VMEM((2, E), jnp.float32)],
    )(xf, W_mem[:2], b_mem[:2], gamma, beta, W_g, b_g, tri)

    counts8 = counts[0].astype(jnp.int32)
    padded = ((counts8 + (RT - 1)) // RT) * RT
    starts8 = jnp.cumsum(padded) - padded
    pos = jnp.arange(NT_G, dtype=jnp.int32) * RT
    gid = jnp.sum(
        (pos[:, None] >= starts8[None, :]).astype(jnp.int32), axis=1) - 1
    starts_s = jnp.broadcast_to(starts8[:, None], (E, 16))

    e_flat = eidx.reshape(A)
    rk_flat = rk.reshape(A)

    mesh = plsc.VectorSubcoreMesh(core_axis_name="c", subcore_axis_name="s")
    sc1 = functools.partial(
        pl.kernel,
        out_type=[jax.ShapeDtypeStruct((A,), jnp.int32),
                  jax.ShapeDtypeStruct((P, Dx), jnp.float32)],
        mesh=mesh,
        scratch_types=[
            pltpu.VMEM((CHUNK,), jnp.int32),
            pltpu.VMEM((CHUNK,), jnp.int32),
            pltpu.VMEM((E, 16), jnp.int32),
            pltpu.VMEM((CHUNK,), jnp.int32),
            pltpu.VMEM((64,), jnp.int32),
            pltpu.VMEM((64,), jnp.int32),
            pltpu.VMEM((64, Dx), jnp.float32),
            pltpu.SemaphoreType.DMA,
        ],
    )(_sc1_body)
    p_of_a, xg = sc1(e_flat, rk_flat, starts_s, xo)

    yg = pl.pallas_call(
        _grouped_body,
        grid_spec=pltpu.PrefetchScalarGridSpec(
            num_scalar_prefetch=1,
            grid=(NT_G,),
            in_specs=[pl.BlockSpec((RT, Dx), lambda i, g: (i, 0)),
                      pl.BlockSpec((1, Dx, Dx), lambda i, g: (g[i], 0, 0)),
                      pl.BlockSpec((1, 1, Dx), lambda i, g: (g[i], 0, 0))],
            out_specs=pl.BlockSpec((RT, Dx), lambda i, g: (i, 0)),
        ),
        out_shape=jax.ShapeDtypeStruct((P, Dx), jnp.float32),
    )(gid, xg, we_b, b_e.reshape(E, 1, Dx))

    sc2 = functools.partial(
        pl.kernel,
        out_type=jax.ShapeDtypeStruct((A, Dx), jnp.float32),
        mesh=mesh,
        scratch_types=[
            pltpu.VMEM((64,), jnp.int32),
            pltpu.VMEM((64, Dx), jnp.float32),
            pltpu.SemaphoreType.DMA,
        ],
    )(_sc2_body)
    link = sc2(yg, p_of_a)

    l2 = link.reshape(N, 2 * Dx)
    out = pl.pallas_call(
        _fit_body,
        grid=(nt,),
        in_specs=[pl.BlockSpec((TILE, 2 * Dx), lambda t: (t, 0)),
                  row, two, full(3, Dx, Dx), full(Dx), full(Dx), full(Dx)],
        out_specs=row,
        out_shape=jax.ShapeDtypeStruct((N, Dx), jnp.float32),
    )(l2, xo, ew, wf3, b_fit, gamma, beta)

    return out.reshape(Bx, Sx, Dx)


# speedup vs baseline: 1.5215x; 1.5215x over previous
import jax
import jax.numpy as jnp
from jax import lax
from jax.experimental import pallas as pl

TILE = 512
D = 1024
E = 8
LN_EPS = 1e-5


def _ln_rows(h, gamma, beta):
    mu = jnp.mean(h, axis=-1, keepdims=True)
    var = jnp.mean((h - mu) ** 2, axis=-1, keepdims=True)
    return gamma * (h - mu) * lax.rsqrt(var + LN_EPS) + beta


def _fused_body(x_ref, wm_ref, bm_ref, g_ref, b_ref, wg_ref, bg_ref,
                we_ref, be_ref, wf_ref, bf_ref, o_ref):
    x = x_ref[...]
    gamma = g_ref[...]
    beta = b_ref[...]
    for c in range(2):
        h = lax.dot_general(x, wm_ref[c], (((1,), (1,)), ((), ())),
                            preferred_element_type=jnp.float32)
        h = h + bm_ref[c][None, :]
        x = x + _ln_rows(h, gamma, beta)

    logits = jnp.dot(x, wg_ref[...], preferred_element_type=jnp.float32)
    logits = logits + bg_ref[...]
    ii = lax.broadcasted_iota(jnp.int32, logits.shape, 1)
    v1 = jnp.max(logits, axis=1, keepdims=True)
    i1 = jnp.min(jnp.where(logits == v1, ii, E), axis=1, keepdims=True)
    sel1 = ii == i1
    l2 = jnp.where(sel1, -jnp.inf, logits)
    v2 = jnp.max(l2, axis=1, keepdims=True)
    i2 = jnp.min(jnp.where(l2 == v2, ii, E), axis=1, keepdims=True)
    sel2 = ii == i2
    g = jnp.exp(v2 - v1)
    w1 = 1.0 / (1.0 + g)
    w2 = g / (1.0 + g)
    c0 = jnp.where(sel1, w1, 0.0)
    c1 = jnp.where(sel2, w2, 0.0)

    xb = x.astype(jnp.bfloat16)
    acc0 = jnp.zeros((TILE, D), jnp.float32)
    acc1 = jnp.zeros((TILE, D), jnp.float32)
    for e in range(E):
        y = lax.dot_general(xb, we_ref[e], (((1,), (1,)), ((), ())),
                            preferred_element_type=jnp.float32)
        acc0 = jnp.where(i1 == e, y, acc0)
        acc1 = jnp.where(i2 == e, y, acc1)
    acc0 = acc0 * w1 + lax.dot_general(c0, be_ref[...],
                                       (((1,), (0,)), ((), ())),
                                       preferred_element_type=jnp.float32)
    acc1 = acc1 * w2 + lax.dot_general(c1, be_ref[...],
                                       (((1,), (0,)), ((), ())),
                                       preferred_element_type=jnp.float32)

    o = lax.dot_general(acc0.astype(jnp.bfloat16), wf_ref[0],
                        (((1,), (0,)), ((), ())),
                        preferred_element_type=jnp.float32)
    o += lax.dot_general(acc1.astype(jnp.bfloat16), wf_ref[1],
                         (((1,), (0,)), ((), ())),
                         preferred_element_type=jnp.float32)
    o += lax.dot_general(xb, wf_ref[2], (((1,), (0,)), ((), ())),
                         preferred_element_type=jnp.float32)
    o += bf_ref[...][None, :]
    o = _ln_rows(o, gamma, beta)
    o_ref[...] = jax.nn.gelu(o, approximate=True)


def kernel(x, W_mem, b_mem, gamma, beta, W_g, b_g, W_e, b_e, W_fit, b_fit,
           choices):
    Bx, Sx, Dx = x.shape
    N = Bx * Sx
    nt = N // TILE
    xf = x.reshape(N, Dx)
    wf3 = W_fit.reshape(3, Dx, Dx).astype(jnp.bfloat16)
    we_b = W_e.astype(jnp.bfloat16)

    full = lambda *s: pl.BlockSpec(s, lambda *_: tuple(0 for _ in s))
    row = pl.BlockSpec((TILE, Dx), lambda t: (t, 0))

    out = pl.pallas_call(
        _fused_body,
        grid=(nt,),
        in_specs=[row,
                  full(2, Dx, Dx), full(2, Dx),
                  full(Dx), full(Dx),
                  full(Dx, E), full(E),
                  full(E, Dx, Dx), full(E, Dx),
                  full(3, Dx, Dx), full(Dx)],
        out_specs=row,
        out_shape=jax.ShapeDtypeStruct((N, Dx), jnp.float32),
    )(xf, W_mem[:2], b_mem[:2], gamma, beta, W_g, b_g,
      we_b, b_e, wf3, b_fit)

    return out.reshape(Bx, Sx, Dx)
